# Initial kernel scaffold; baseline (speedup 1.0000x reference)
import jax
import jax.numpy as jnp
from jax.experimental import pallas as pl


def kernel(x, edge_index, W1, b1, W2, b2, Wfc, bfc):
    raise NotImplementedError("write your pallas kernel here")


# speedup vs baseline: 8.6616x; 8.6616x over previous
import functools

import jax
import jax.numpy as jnp
from jax import lax
from jax.experimental import pallas as pl
from jax.experimental.pallas import tpu as pltpu
from jax.experimental.pallas import tpu_sc as plsc

N_NODES = 10000
N_EDGES = 320000
D_IN = 128
D_HID = 256
D_OUT = 128

NC = 2
NS = 16
NW = NC * NS

K = 128
BPT = 80
EPT = K * BPT
E_PAD = NW * EPT
ROWS_PER_TILE = 632
N_PAD = NS * ROWS_PER_TILE
TRASH = N_NODES

RB = 1000
GRID = N_NODES // RB

_mesh = plsc.VectorSubcoreMesh(core_axis_name="c", subcore_axis_name="s")


DW = 16


@functools.partial(
    pl.kernel,
    mesh=_mesh,
    out_type=jax.ShapeDtypeStruct((NC, N_PAD, DW), jnp.float32),
    scratch_types=[
        pltpu.VMEM((BPT, K), jnp.int32),
        pltpu.VMEM((K, DW), jnp.float32),
        pltpu.VMEM_SHARED((N_PAD, DW), jnp.float32),
    ],
)
def _sc_degree(dst_hbm, deg_out, dstv, buf, zsh):
    c = lax.axis_index("c")
    s = lax.axis_index("s")
    wid = s * NC + c
    r0 = s * ROWS_PER_TILE

    pltpu.sync_copy(dst_hbm.at[pl.ds(wid * BPT, BPT)], dstv)

    zeros16 = jnp.zeros((DW,), jnp.float32)
    ones16 = jnp.ones((DW,), jnp.float32)

    def _zero(i, _):
        buf[i, pl.ds(0, DW)] = zeros16
        return _

    lax.fori_loop(0, K, _zero, 0)

    for t in range(5):
        rows = 128 if t < 4 else ROWS_PER_TILE - 4 * 128
        pltpu.sync_copy(buf.at[pl.ds(0, rows)],
                        zsh.at[pl.ds(r0 + t * 128, rows)])

    def _fill(i, _):
        buf[i, pl.ds(0, DW)] = ones16
        return _

    lax.fori_loop(0, K, _fill, 0)

    plsc.subcore_barrier()

    def _hist(j, _):
        pltpu.sync_copy(buf, zsh.at[dstv.at[j]], add=True)
        return _

    lax.fori_loop(0, BPT, _hist, 0)

    plsc.subcore_barrier()

    for t in range(5):
        rows = 128 if t < 4 else ROWS_PER_TILE - 4 * 128
        pltpu.sync_copy(zsh.at[pl.ds(r0 + t * 128, rows)],
                        buf.at[pl.ds(0, rows)])
        pltpu.sync_copy(buf.at[pl.ds(0, rows)],
                        deg_out.at[c, pl.ds(r0 + t * 128, rows)])


@functools.partial(
    pl.kernel,
    mesh=_mesh,
    out_type=jax.ShapeDtypeStruct((NC, N_PAD, D_IN), jnp.float32),
    scratch_types=[
        pltpu.VMEM((BPT, K), jnp.int32),
        pltpu.VMEM((BPT, K), jnp.int32),
        pltpu.VMEM((K, D_IN), jnp.float32),
        pltpu.VMEM_SHARED((N_PAD, D_IN), jnp.float32),
        pltpu.SemaphoreType.DMA,
    ],
)
def _sc_scatter(u_hbm, src_hbm, dst_hbm, z_out, srcv, dstv, buf, zsh, sem):
    c = lax.axis_index("c")
    s = lax.axis_index("s")
    wid = s * NC + c
    r0 = s * ROWS_PER_TILE

    pltpu.sync_copy(src_hbm.at[pl.ds(wid * BPT, BPT)], srcv)
    pltpu.sync_copy(dst_hbm.at[pl.ds(wid * BPT, BPT)], dstv)

    zeros16 = jnp.zeros((16,), jnp.float32)

    def _zero(i, _):
        j = i // (D_IN // 16)
        k = i % (D_IN // 16)
        buf[j, pl.ds(k * 16, 16)] = zeros16
        return _

    lax.fori_loop(0, K * D_IN // 16, _zero, 0)

    for t in range(5):
        rows = 128 if t < 4 else ROWS_PER_TILE - 4 * 128
        pltpu.sync_copy(buf.at[pl.ds(0, rows)],
                        zsh.at[pl.ds(r0 + t * 128, rows)])

    plsc.subcore_barrier()

    def _edge(j, _):
        pltpu.async_copy(u_hbm.at[srcv.at[j]], buf, sem).wait()
        pltpu.sync_copy(buf, zsh.at[dstv.at[j]], add=True)
        return _

    lax.fori_loop(0, BPT, _edge, 0)

    plsc.subcore_barrier()

    for t in range(5):
        rows = 128 if t < 4 else ROWS_PER_TILE - 4 * 128
        pltpu.sync_copy(zsh.at[pl.ds(r0 + t * 128, rows)],
                        buf.at[pl.ds(0, rows)])
        pltpu.sync_copy(buf.at[pl.ds(0, rows)],
                        z_out.at[c, pl.ds(r0 + t * 128, rows)])


def _tc_prep_body(d0_ref, d1_ref, x_ref, u1_ref, dinv_ref):
    deg = d0_ref[:, :1] + d1_ref[:, :1] + 1.0
    dv = lax.rsqrt(deg)
    dinv_ref[...] = dv
    u1_ref[...] = x_ref[...] * dv


def _tc_prep(deg0, deg1, x):
    return pl.pallas_call(
        _tc_prep_body,
        grid=(GRID,),
        in_specs=[
            pl.BlockSpec((RB, DW), lambda i: (i, 0)),
            pl.BlockSpec((RB, DW), lambda i: (i, 0)),
            pl.BlockSpec((RB, D_IN), lambda i: (i, 0)),
        ],
        out_specs=[
            pl.BlockSpec((RB, D_IN), lambda i: (i, 0)),
            pl.BlockSpec((RB, 1), lambda i: (i, 0)),
        ],
        out_shape=[
            jax.ShapeDtypeStruct((N_NODES, D_IN), jnp.float32),
            jax.ShapeDtypeStruct((N_NODES, 1), jnp.float32),
        ],
    )(deg0, deg1, x)


def _tc_mid_body(z0_ref, z1_ref, u1_ref, dinv_ref, w1_ref, b1_ref, w2_ref,
                 u2_ref):
    dv = dinv_ref[...]
    agg = dv * (z0_ref[...] + z1_ref[...] + u1_ref[...])
    h1 = jnp.maximum(
        jnp.dot(agg, w1_ref[...], preferred_element_type=jnp.float32)
        + b1_ref[...], 0.0)
    m2 = jnp.dot(h1, w2_ref[...], preferred_element_type=jnp.float32)
    u2_ref[...] = dv * m2


def _tc_mid(z0, z1, u1, dinv, W1, b1r, W2):
    return pl.pallas_call(
        _tc_mid_body,
        grid=(GRID,),
        in_specs=[
            pl.BlockSpec((RB, D_IN), lambda i: (i, 0)),
            pl.BlockSpec((RB, D_IN), lambda i: (i, 0)),
            pl.BlockSpec((RB, D_IN), lambda i: (i, 0)),
            pl.BlockSpec((RB, 1), lambda i: (i, 0)),
            pl.BlockSpec((D_IN, D_HID), lambda i: (0, 0)),
            pl.BlockSpec((1, D_HID), lambda i: (0, 0)),
            pl.BlockSpec((D_HID, D_OUT), lambda i: (0, 0)),
        ],
        out_specs=pl.BlockSpec((RB, D_OUT), lambda i: (i, 0)),
        out_shape=jax.ShapeDtypeStruct((N_NODES, D_OUT), jnp.float32),
    )(z0, z1, u1, dinv, W1, b1r, W2)


def _tc_final_body(z0_ref, z1_ref, u2_ref, dinv_ref, wfc_ref, cb_ref, out_ref):
    agg = dinv_ref[...] * (z0_ref[...] + z1_ref[...] + u2_ref[...])
    out_ref[...] = (
        jnp.dot(agg, wfc_ref[...], preferred_element_type=jnp.float32)
        + cb_ref[...])


def _tc_final(z0, z1, u2, dinv, Wfc, cbias):
    return pl.pallas_call(
        _tc_final_body,
        grid=(GRID,),
        in_specs=[
            pl.BlockSpec((RB, D_OUT), lambda i: (i, 0)),
            pl.BlockSpec((RB, D_OUT), lambda i: (i, 0)),
            pl.BlockSpec((RB, D_OUT), lambda i: (i, 0)),
            pl.BlockSpec((RB, 1), lambda i: (i, 0)),
            pl.BlockSpec((D_OUT, D_IN), lambda i: (0, 0)),
            pl.BlockSpec((1, D_IN), lambda i: (0, 0)),
        ],
        out_specs=pl.BlockSpec((RB, D_IN), lambda i: (i, 0)),
        out_shape=jax.ShapeDtypeStruct((N_NODES, D_IN), jnp.float32),
    )(z0, z1, u2, dinv, Wfc, cbias)


def _tc_bias_body(b2_ref, wfc_ref, bfc_ref, cb_ref):
    cb_ref[...] = (
        jnp.dot(b2_ref[...], wfc_ref[...], preferred_element_type=jnp.float32)
        + bfc_ref[...])


def _tc_bias(b2r, Wfc, bfcr):
    return pl.pallas_call(
        _tc_bias_body,
        out_shape=jax.ShapeDtypeStruct((1, D_IN), jnp.float32),
    )(b2r, Wfc, bfcr)


def kernel(x, edge_index, W1, b1, W2, b2, Wfc, bfc):
    src = edge_index[0].astype(jnp.int32)
    dst = edge_index[1].astype(jnp.int32)
    pad = E_PAD - N_EDGES
    src_p = jnp.concatenate(
        [src, jnp.zeros((pad,), jnp.int32)]).reshape(E_PAD // K, K)
    dst_p = jnp.concatenate(
        [dst, jnp.full((pad,), TRASH, jnp.int32)]).reshape(E_PAD // K, K)

    deg_parts = _sc_degree(dst_p)
    u1, dinv = _tc_prep(deg_parts[0, :N_NODES], deg_parts[1, :N_NODES], x)

    z1 = _sc_scatter(u1, src_p, dst_p)
    cbias = _tc_bias(b2.reshape(1, D_OUT), Wfc, bfc.reshape(1, D_IN))
    u2 = _tc_mid(z1[0], z1[1], u1, dinv, W1, b1.reshape(1, D_HID), W2)

    z2 = _sc_scatter(u2, src_p, dst_p)
    out = _tc_final(z2[0], z2[1], u2, dinv, Wfc, cbias)
    return out


# speedup vs baseline: 23.5637x; 2.7205x over previous
import functools

import jax
import jax.numpy as jnp
from jax import lax
from jax.experimental import pallas as pl
from jax.experimental.pallas import tpu as pltpu
from jax.experimental.pallas import tpu_sc as plsc

N_NODES = 10000
N_EDGES = 320000
D_IN = 128
D_HID = 256
D_OUT = 128

NC = 2
NS = 16
NW = NC * NS

K = 128
BPT = 80
EPT = K * BPT
E_PAD = NW * EPT
ROWS_PER_TILE = 632
N_PAD = NS * ROWS_PER_TILE
TRASH = N_NODES

RB = 1000
GRID = N_NODES // RB

_mesh = plsc.VectorSubcoreMesh(core_axis_name="c", subcore_axis_name="s")


DW = 16


@functools.partial(
    pl.kernel,
    mesh=_mesh,
    out_type=jax.ShapeDtypeStruct((NC, N_PAD, DW), jnp.float32),
    scratch_types=[
        pltpu.VMEM((BPT, K), jnp.int32),
        pltpu.VMEM((K, DW), jnp.float32),
        pltpu.VMEM_SHARED((N_PAD, DW), jnp.float32),
    ],
)
def _sc_degree(dst_hbm, deg_out, dstv, buf, zsh):
    c = lax.axis_index("c")
    s = lax.axis_index("s")
    wid = s * NC + c
    r0 = s * ROWS_PER_TILE

    pltpu.sync_copy(dst_hbm.at[pl.ds(wid * BPT, BPT)], dstv)

    zeros16 = jnp.zeros((DW,), jnp.float32)
    ones16 = jnp.ones((DW,), jnp.float32)

    def _zero(i, _):
        buf[i, pl.ds(0, DW)] = zeros16
        return _

    lax.fori_loop(0, K, _zero, 0)

    for t in range(5):
        rows = 128 if t < 4 else ROWS_PER_TILE - 4 * 128
        pltpu.sync_copy(buf.at[pl.ds(0, rows)],
                        zsh.at[pl.ds(r0 + t * 128, rows)])

    def _fill(i, _):
        buf[i, pl.ds(0, DW)] = ones16
        return _

    lax.fori_loop(0, K, _fill, 0)

    plsc.subcore_barrier()

    def _hist(j, _):
        pltpu.sync_copy(buf, zsh.at[dstv.at[j]], add=True)
        return _

    lax.fori_loop(0, BPT, _hist, 0)

    plsc.subcore_barrier()

    for t in range(5):
        rows = 128 if t < 4 else ROWS_PER_TILE - 4 * 128
        pltpu.sync_copy(zsh.at[pl.ds(r0 + t * 128, rows)],
                        buf.at[pl.ds(0, rows)])
        pltpu.sync_copy(buf.at[pl.ds(0, rows)],
                        deg_out.at[c, pl.ds(r0 + t * 128, rows)])


@functools.partial(
    pl.kernel,
    mesh=_mesh,
    out_type=jax.ShapeDtypeStruct((NC, N_PAD, D_IN), jnp.float32),
    scratch_types=[
        pltpu.VMEM((BPT, K), jnp.int32),
        pltpu.VMEM((BPT, K), jnp.int32),
        pltpu.VMEM((K, D_IN), jnp.float32),
        pltpu.VMEM_SHARED((N_PAD, D_IN), jnp.float32),
        pltpu.SemaphoreType.DMA,
    ],
)
def _sc_scatter(u_hbm, src_hbm, dst_hbm, z_out, srcv, dstv, buf, zsh, sem):
    c = lax.axis_index("c")
    s = lax.axis_index("s")
    wid = s * NC + c
    r0 = s * ROWS_PER_TILE

    pltpu.sync_copy(src_hbm.at[pl.ds(wid * BPT, BPT)], srcv)
    pltpu.sync_copy(dst_hbm.at[pl.ds(wid * BPT, BPT)], dstv)

    zeros16 = jnp.zeros((16,), jnp.float32)

    def _zero(i, _):
        j = i // (D_IN // 16)
        k = i % (D_IN // 16)
        buf[j, pl.ds(k * 16, 16)] = zeros16
        return _

    lax.fori_loop(0, K * D_IN // 16, _zero, 0)

    for t in range(5):
        rows = 128 if t < 4 else ROWS_PER_TILE - 4 * 128
        pltpu.sync_copy(buf.at[pl.ds(0, rows)],
                        zsh.at[pl.ds(r0 + t * 128, rows)])

    plsc.subcore_barrier()

    def _edge(j, _):
        pltpu.async_copy(u_hbm.at[srcv.at[j]], buf, sem).wait()
        pltpu.sync_copy(buf, zsh.at[dstv.at[j]], add=True)
        return _

    lax.fori_loop(0, BPT, _edge, 0)

    plsc.subcore_barrier()

    for t in range(5):
        rows = 128 if t < 4 else ROWS_PER_TILE - 4 * 128
        pltpu.sync_copy(zsh.at[pl.ds(r0 + t * 128, rows)],
                        buf.at[pl.ds(0, rows)])
        pltpu.sync_copy(buf.at[pl.ds(0, rows)],
                        z_out.at[c, pl.ds(r0 + t * 128, rows)])


def _tc_prep_body(d0_ref, d1_ref, x_ref, u1_ref, dinv_ref):
    deg = d0_ref[:, :1] + d1_ref[:, :1] + 1.0
    dv = lax.rsqrt(deg)
    dinv_ref[...] = dv
    u1_ref[...] = x_ref[...] * dv


def _tc_prep(deg0, deg1, x):
    return pl.pallas_call(
        _tc_prep_body,
        grid=(GRID,),
        in_specs=[
            pl.BlockSpec((RB, DW), lambda i: (i, 0)),
            pl.BlockSpec((RB, DW), lambda i: (i, 0)),
            pl.BlockSpec((RB, D_IN), lambda i: (i, 0)),
        ],
        out_specs=[
            pl.BlockSpec((RB, D_IN), lambda i: (i, 0)),
            pl.BlockSpec((RB, 1), lambda i: (i, 0)),
        ],
        out_shape=[
            jax.ShapeDtypeStruct((N_NODES, D_IN), jnp.float32),
            jax.ShapeDtypeStruct((N_NODES, 1), jnp.float32),
        ],
    )(deg0, deg1, x)


def _tc_mid_body(z0_ref, z1_ref, u1_ref, dinv_ref, w1_ref, b1_ref, w2_ref,
                 u2_ref):
    dv = dinv_ref[...]
    agg = dv * (z0_ref[...] + z1_ref[...] + u1_ref[...])
    h1 = jnp.maximum(
        jnp.dot(agg, w1_ref[...], preferred_element_type=jnp.float32)
        + b1_ref[...], 0.0)
    m2 = jnp.dot(h1, w2_ref[...], preferred_element_type=jnp.float32)
    u2_ref[...] = dv * m2


def _tc_mid(z0, z1, u1, dinv, W1, b1r, W2):
    return pl.pallas_call(
        _tc_mid_body,
        grid=(GRID,),
        in_specs=[
            pl.BlockSpec((RB, D_IN), lambda i: (i, 0)),
            pl.BlockSpec((RB, D_IN), lambda i: (i, 0)),
            pl.BlockSpec((RB, D_IN), lambda i: (i, 0)),
            pl.BlockSpec((RB, 1), lambda i: (i, 0)),
            pl.BlockSpec((D_IN, D_HID), lambda i: (0, 0)),
            pl.BlockSpec((1, D_HID), lambda i: (0, 0)),
            pl.BlockSpec((D_HID, D_OUT), lambda i: (0, 0)),
        ],
        out_specs=pl.BlockSpec((RB, D_OUT), lambda i: (i, 0)),
        out_shape=jax.ShapeDtypeStruct((N_NODES, D_OUT), jnp.float32),
    )(z0, z1, u1, dinv, W1, b1r, W2)


def _tc_final_body(z0_ref, z1_ref, u2_ref, dinv_ref, wfc_ref, cb_ref, out_ref):
    agg = dinv_ref[...] * (z0_ref[...] + z1_ref[...] + u2_ref[...])
    out_ref[...] = (
        jnp.dot(agg, wfc_ref[...], preferred_element_type=jnp.float32)
        + cb_ref[...])


def _tc_final(z0, z1, u2, dinv, Wfc, cbias):
    return pl.pallas_call(
        _tc_final_body,
        grid=(GRID,),
        in_specs=[
            pl.BlockSpec((RB, D_OUT), lambda i: (i, 0)),
            pl.BlockSpec((RB, D_OUT), lambda i: (i, 0)),
            pl.BlockSpec((RB, D_OUT), lambda i: (i, 0)),
            pl.BlockSpec((RB, 1), lambda i: (i, 0)),
            pl.BlockSpec((D_OUT, D_IN), lambda i: (0, 0)),
            pl.BlockSpec((1, D_IN), lambda i: (0, 0)),
        ],
        out_specs=pl.BlockSpec((RB, D_IN), lambda i: (i, 0)),
        out_shape=jax.ShapeDtypeStruct((N_NODES, D_IN), jnp.float32),
    )(z0, z1, u2, dinv, Wfc, cbias)


def _tc_bias_body(b2_ref, wfc_ref, bfc_ref, cb_ref):
    cb_ref[...] = (
        jnp.dot(b2_ref[...], wfc_ref[...], preferred_element_type=jnp.float32)
        + bfc_ref[...])


def _tc_bias(b2r, Wfc, bfcr):
    return pl.pallas_call(
        _tc_bias_body,
        out_shape=jax.ShapeDtypeStruct((1, D_IN), jnp.float32),
    )(b2r, Wfc, bfcr)


def kernel(x, edge_index, W1, b1, W2, b2, Wfc, bfc):
    src = edge_index[0].astype(jnp.int32)
    dst = edge_index[1].astype(jnp.int32)
    pad = E_PAD - N_EDGES
    pad_ids = jnp.arange(pad, dtype=jnp.int32)
    src_p = jnp.concatenate(
        [src, pad_ids % N_NODES]).reshape(E_PAD // K, K)
    dst_p = jnp.concatenate(
        [dst, TRASH + pad_ids % (N_PAD - N_NODES)]).reshape(E_PAD // K, K)

    deg_parts = _sc_degree(dst_p)
    u1, dinv = _tc_prep(deg_parts[0, :N_NODES], deg_parts[1, :N_NODES], x)

    z1 = _sc_scatter(u1, src_p, dst_p)
    cbias = _tc_bias(b2.reshape(1, D_OUT), Wfc, bfc.reshape(1, D_IN))
    u2 = _tc_mid(z1[0], z1[1], u1, dinv, W1, b1.reshape(1, D_HID), W2)

    z2 = _sc_scatter(u2, src_p, dst_p)
    out = _tc_final(z2[0], z2[1], u2, dinv, Wfc, cbias)
    return out


# speedup vs baseline: 32.8608x; 1.3946x over previous
import functools

import jax
import jax.numpy as jnp
from jax import lax
from jax.experimental import pallas as pl
from jax.experimental.pallas import tpu as pltpu
from jax.experimental.pallas import tpu_sc as plsc

N_NODES = 10000
N_EDGES = 320000
D_IN = 128
D_HID = 256
D_OUT = 128

NC = 2
NS = 16
NW = NC * NS

K = 128
BPT = 80
EPT = K * BPT
E_PAD = NW * EPT
ROWS_PER_TILE = 632
N_PAD = NS * ROWS_PER_TILE
TRASH = N_NODES

RB = 1000
GRID = N_NODES // RB

_mesh = plsc.VectorSubcoreMesh(core_axis_name="c", subcore_axis_name="s")


DW = 16


@functools.partial(
    pl.kernel,
    mesh=_mesh,
    out_type=jax.ShapeDtypeStruct((NC, N_PAD, DW), jnp.float32),
    scratch_types=[
        pltpu.VMEM((BPT, K), jnp.int32),
        pltpu.VMEM((K, DW), jnp.float32),
        pltpu.VMEM_SHARED((N_PAD, DW), jnp.float32),
    ],
)
def _sc_degree(dst_hbm, deg_out, dstv, buf, zsh):
    c = lax.axis_index("c")
    s = lax.axis_index("s")
    wid = s * NC + c
    r0 = s * ROWS_PER_TILE

    pltpu.sync_copy(dst_hbm.at[pl.ds(wid * BPT, BPT)], dstv)

    zeros16 = jnp.zeros((DW,), jnp.float32)
    ones16 = jnp.ones((DW,), jnp.float32)

    def _zero(i, _):
        buf[i, pl.ds(0, DW)] = zeros16
        return _

    lax.fori_loop(0, K, _zero, 0)

    for t in range(5):
        rows = 128 if t < 4 else ROWS_PER_TILE - 4 * 128
        pltpu.sync_copy(buf.at[pl.ds(0, rows)],
                        zsh.at[pl.ds(r0 + t * 128, rows)])

    def _fill(i, _):
        buf[i, pl.ds(0, DW)] = ones16
        return _

    lax.fori_loop(0, K, _fill, 0)

    plsc.subcore_barrier()

    def _hist(j, _):
        pltpu.sync_copy(buf, zsh.at[dstv.at[j]], add=True)
        return _

    lax.fori_loop(0, BPT, _hist, 0)

    plsc.subcore_barrier()

    for t in range(5):
        rows = 128 if t < 4 else ROWS_PER_TILE - 4 * 128
        pltpu.sync_copy(zsh.at[pl.ds(r0 + t * 128, rows)],
                        buf.at[pl.ds(0, rows)])
        pltpu.sync_copy(buf.at[pl.ds(0, rows)],
                        deg_out.at[c, pl.ds(r0 + t * 128, rows)])


@functools.partial(
    pl.kernel,
    mesh=_mesh,
    out_type=jax.ShapeDtypeStruct((NC, N_PAD, D_IN), jnp.float32),
    scratch_types=[
        pltpu.VMEM((BPT // 2, K), jnp.int32),
        pltpu.VMEM((BPT // 2, K), jnp.int32),
        pltpu.VMEM((K, D_IN), jnp.float32),
        pltpu.VMEM((K, D_IN), jnp.float32),
        pltpu.VMEM_SHARED((N_PAD, D_IN), jnp.float32),
        pltpu.SemaphoreType.DMA,
        pltpu.SemaphoreType.DMA,
    ],
)
def _sc_scatter(u_hbm, src_hbm, dst_hbm, z_out, srcv, dstv, buf, buf2, zsh,
                sem, sem2):
    c = lax.axis_index("c")
    s = lax.axis_index("s")
    wid = s * NC + c
    r0 = s * ROWS_PER_TILE
    BPH = BPT // 2

    zeros16 = jnp.zeros((16,), jnp.float32)

    def _zero(i, _):
        j = i // (D_IN // 16)
        k = i % (D_IN // 16)
        buf[j, pl.ds(k * 16, 16)] = zeros16
        return _

    lax.fori_loop(0, K * D_IN // 16, _zero, 0)

    for t in range(5):
        rows = 128 if t < 4 else ROWS_PER_TILE - 4 * 128
        pltpu.sync_copy(buf.at[pl.ds(0, rows)],
                        zsh.at[pl.ds(r0 + t * 128, rows)])

    plsc.subcore_barrier()

    for h in range(2):
        b0 = wid * BPT + h * BPH
        pltpu.sync_copy(src_hbm.at[pl.ds(b0, BPH)], srcv)
        pltpu.sync_copy(dst_hbm.at[pl.ds(b0, BPH)], dstv)

        pltpu.async_copy(u_hbm.at[srcv.at[0]], buf, sem)

        def _pair(i, _):
            g = i * 2
            pltpu.async_copy(u_hbm.at[srcv.at[g + 1]], buf2, sem2)
            pltpu.make_async_copy(u_hbm.at[srcv.at[g]], buf, sem).wait()
            pltpu.sync_copy(buf, zsh.at[dstv.at[g]], add=True)

            @pl.when(g + 2 < BPH)
            def _start_next():
                pltpu.async_copy(u_hbm.at[srcv.at[g + 2]], buf, sem)

            pltpu.make_async_copy(u_hbm.at[srcv.at[g + 1]], buf2, sem2).wait()
            pltpu.sync_copy(buf2, zsh.at[dstv.at[g + 1]], add=True)
            return _

        lax.fori_loop(0, BPH // 2, _pair, 0)

    plsc.subcore_barrier()

    for t in range(5):
        rows = 128 if t < 4 else ROWS_PER_TILE - 4 * 128
        pltpu.sync_copy(zsh.at[pl.ds(r0 + t * 128, rows)],
                        buf.at[pl.ds(0, rows)])
        pltpu.sync_copy(buf.at[pl.ds(0, rows)],
                        z_out.at[c, pl.ds(r0 + t * 128, rows)])


def _tc_prep_body(d0_ref, d1_ref, x_ref, u1_ref, dinv_ref):
    deg = d0_ref[:, :1] + d1_ref[:, :1] + 1.0
    dv = lax.rsqrt(deg)
    dinv_ref[...] = dv
    u1_ref[...] = x_ref[...] * dv


def _tc_prep(deg0, deg1, x):
    return pl.pallas_call(
        _tc_prep_body,
        grid=(GRID,),
        in_specs=[
            pl.BlockSpec((RB, DW), lambda i: (i, 0)),
            pl.BlockSpec((RB, DW), lambda i: (i, 0)),
            pl.BlockSpec((RB, D_IN), lambda i: (i, 0)),
        ],
        out_specs=[
            pl.BlockSpec((RB, D_IN), lambda i: (i, 0)),
            pl.BlockSpec((RB, 1), lambda i: (i, 0)),
        ],
        out_shape=[
            jax.ShapeDtypeStruct((N_NODES, D_IN), jnp.float32),
            jax.ShapeDtypeStruct((N_NODES, 1), jnp.float32),
        ],
    )(deg0, deg1, x)


def _tc_mid_body(z0_ref, z1_ref, u1_ref, dinv_ref, w1_ref, b1_ref, w2_ref,
                 u2_ref):
    dv = dinv_ref[...]
    agg = dv * (z0_ref[...] + z1_ref[...] + u1_ref[...])
    h1 = jnp.maximum(
        jnp.dot(agg, w1_ref[...], preferred_element_type=jnp.float32)
        + b1_ref[...], 0.0)
    m2 = jnp.dot(h1, w2_ref[...], preferred_element_type=jnp.float32)
    u2_ref[...] = dv * m2


def _tc_mid(z0, z1, u1, dinv, W1, b1r, W2):
    return pl.pallas_call(
        _tc_mid_body,
        grid=(GRID,),
        in_specs=[
            pl.BlockSpec((RB, D_IN), lambda i: (i, 0)),
            pl.BlockSpec((RB, D_IN), lambda i: (i, 0)),
            pl.BlockSpec((RB, D_IN), lambda i: (i, 0)),
            pl.BlockSpec((RB, 1), lambda i: (i, 0)),
            pl.BlockSpec((D_IN, D_HID), lambda i: (0, 0)),
            pl.BlockSpec((1, D_HID), lambda i: (0, 0)),
            pl.BlockSpec((D_HID, D_OUT), lambda i: (0, 0)),
        ],
        out_specs=pl.BlockSpec((RB, D_OUT), lambda i: (i, 0)),
        out_shape=jax.ShapeDtypeStruct((N_NODES, D_OUT), jnp.float32),
    )(z0, z1, u1, dinv, W1, b1r, W2)


def _tc_final_body(z0_ref, z1_ref, u2_ref, dinv_ref, wfc_ref, cb_ref, out_ref):
    agg = dinv_ref[...] * (z0_ref[...] + z1_ref[...] + u2_ref[...])
    out_ref[...] = (
        jnp.dot(agg, wfc_ref[...], preferred_element_type=jnp.float32)
        + cb_ref[...])


def _tc_final(z0, z1, u2, dinv, Wfc, cbias):
    return pl.pallas_call(
        _tc_final_body,
        grid=(GRID,),
        in_specs=[
            pl.BlockSpec((RB, D_OUT), lambda i: (i, 0)),
            pl.BlockSpec((RB, D_OUT), lambda i: (i, 0)),
            pl.BlockSpec((RB, D_OUT), lambda i: (i, 0)),
            pl.BlockSpec((RB, 1), lambda i: (i, 0)),
            pl.BlockSpec((D_OUT, D_IN), lambda i: (0, 0)),
            pl.BlockSpec((1, D_IN), lambda i: (0, 0)),
        ],
        out_specs=pl.BlockSpec((RB, D_IN), lambda i: (i, 0)),
        out_shape=jax.ShapeDtypeStruct((N_NODES, D_IN), jnp.float32),
    )(z0, z1, u2, dinv, Wfc, cbias)


def _tc_bias_body(b2_ref, wfc_ref, bfc_ref, cb_ref):
    cb_ref[...] = (
        jnp.dot(b2_ref[...], wfc_ref[...], preferred_element_type=jnp.float32)
        + bfc_ref[...])


def _tc_bias(b2r, Wfc, bfcr):
    return pl.pallas_call(
        _tc_bias_body,
        out_shape=jax.ShapeDtypeStruct((1, D_IN), jnp.float32),
    )(b2r, Wfc, bfcr)


def kernel(x, edge_index, W1, b1, W2, b2, Wfc, bfc):
    src = edge_index[0].astype(jnp.int32)
    dst = edge_index[1].astype(jnp.int32)
    pad = E_PAD - N_EDGES
    pad_ids = jnp.arange(pad, dtype=jnp.int32)
    src_p = jnp.concatenate(
        [src, pad_ids % N_NODES]).reshape(E_PAD // K, K)
    dst_p = jnp.concatenate(
        [dst, TRASH + pad_ids % (N_PAD - N_NODES)]).reshape(E_PAD // K, K)

    deg_parts = _sc_degree(dst_p)
    u1, dinv = _tc_prep(deg_parts[0, :N_NODES], deg_parts[1, :N_NODES], x)

    z1 = _sc_scatter(u1, src_p, dst_p)
    cbias = _tc_bias(b2.reshape(1, D_OUT), Wfc, bfc.reshape(1, D_IN))
    u2 = _tc_mid(z1[0], z1[1], u1, dinv, W1, b1.reshape(1, D_HID), W2)

    z2 = _sc_scatter(u2, src_p, dst_p)
    out = _tc_final(z2[0], z2[1], u2, dinv, Wfc, cbias)
    return out
